# Initial kernel scaffold; baseline (speedup 1.0000x reference)
import jax
import jax.numpy as jnp
from jax.experimental import pallas as pl


def kernel(x):
    raise NotImplementedError("write your pallas kernel here")


# speedup vs baseline: 3.2885x; 3.2885x over previous
import jax
import jax.numpy as jnp
from jax.experimental import pallas as pl
from jax.experimental.pallas import tpu as pltpu

_K = 64
_N = 32768
_R = 8


def _body(x_ref, o_ref, key_ref):
    x = x_ref[...]
    u = jax.lax.bitcast_convert_type(x, jnp.uint32)
    neg = u >> jnp.uint32(31)
    key = u ^ (neg * jnp.uint32(0x7FFFFFFF) + jnp.uint32(0x80000000))
    key_ref[...] = key

    kK = jnp.int32(_K)

    def val_step(i, t):
        b = 31 - i
        cand = t | (jnp.uint32(1) << b.astype(jnp.uint32))
        cnt = jnp.sum((key_ref[...] >= cand[:, None]).astype(jnp.int32), axis=1)
        return jnp.where(cnt >= kK, cand, t)

    t0 = jnp.zeros((_R,), jnp.uint32)
    t = jax.lax.fori_loop(0, 32, val_step, t0)

    keyv = key_ref[...]
    gt = keyv > t[:, None]
    eq = keyv == t[:, None]
    cg = jnp.sum(gt.astype(jnp.int32), axis=1)
    need = kK - cg

    col = jax.lax.broadcasted_iota(jnp.int32, (_R, _N), 1)

    def col_step(i, L):
        b = 14 - i
        cand = L | (jnp.int32(1) << b
                    )
        cnt = jnp.sum((eq & (col < cand[:, None])).astype(jnp.int32), axis=1)
        return jnp.where(cnt <= need - 1, cand, L)

    L0 = jnp.zeros((_R,), jnp.int32)
    L = jax.lax.fori_loop(0, 15, col_step, L0)

    mask = gt | (eq & (col <= L[:, None]))
    o_ref[...] = jnp.where(mask, x, 0.0)


def kernel(x):
    return pl.pallas_call(
        _body,
        grid=(x.shape[0] // _R,),
        in_specs=[pl.BlockSpec((_R, _N), lambda i: (i, 0))],
        out_specs=pl.BlockSpec((_R, _N), lambda i: (i, 0)),
        out_shape=jax.ShapeDtypeStruct(x.shape, x.dtype),
        scratch_shapes=[pltpu.VMEM((_R, _N), jnp.uint32)],
    )(x)


# speedup vs baseline: 8.0498x; 2.4479x over previous
import functools

import jax
import jax.numpy as jnp
from jax import lax
from jax.experimental import pallas as pl
from jax.experimental.pallas import tpu as pltpu
from jax.experimental.pallas import tpu_sc as plsc

_K = 64
_N = 32768
_ROWS = 128
_NC = 2
_NS = 16
_NW = _NC * _NS
_RPW = _ROWS // _NW
_NBV = _N // 256
_NSV = _NBV // 16
_SVCAP = 2048
_CCAP = 4096


def _keyify(v):
    u = lax.bitcast_convert_type(v, jnp.uint32)
    return u ^ ((u >> jnp.uint32(31)) * jnp.uint32(0x7FFFFFFF)
                + jnp.uint32(0x80000000))


def _unkey(key):
    pos = key >> jnp.uint32(31)
    u = key ^ (jnp.uint32(0x80000000)
               + (jnp.uint32(1) - pos) * jnp.uint32(0x7FFFFFFF))
    return lax.bitcast_convert_type(u, jnp.float32)


def _count(m):
    return jnp.sum(jnp.where(m, 1, 0))


def _select_kth_key(key_ref, nv, k):
    def bit_step(i, t):
        sh = (jnp.uint32(31) - i.astype(jnp.uint32))
        cand = t | jnp.full((16,), jnp.uint32(1) << sh, jnp.uint32)

        def cbody(j, cnt):
            kv = key_ref[pl.ds(j * 16, 16)]
            return cnt + jnp.where(kv >= cand, 1, 0)

        cnt = lax.fori_loop(0, nv, cbody, jnp.zeros((16,), jnp.int32))
        return jnp.where(jnp.sum(cnt) >= k, cand, t)

    return lax.fori_loop(0, 32, bit_step, jnp.zeros((16,), jnp.uint32))


def _body(x_hbm, o_hbm, rowbuf, zbuf, bmax, smax, skey, sv_val, sv_id, sv_key,
          s2_id, c_val, c_idx, c_key, eq_idx, st_val, st_idx, k_val, k_idx,
          sem):
    wid = lax.axis_index("s") * _NC + lax.axis_index("c")
    iota = jnp.arange(16, dtype=jnp.int32)
    zero16f = jnp.zeros((16,), jnp.float32)

    def zinit(i, _):
        zbuf[pl.ds(i * 16, 16)] = zero16f
        return 0

    lax.fori_loop(0, _N // 16, zinit, 0)

    def do_row(r, _):
        row = wid * _RPW + r
        pltpu.sync_copy(x_hbm.at[pl.ds(row * _N, _N)], rowbuf)

        def gbody(g, _):
            base = g * 256
            m = rowbuf[pl.ds(base, 16)]
            for j in range(1, 16):
                m = jnp.maximum(m, rowbuf[pl.ds(base + 16 * j, 16)])
            bmax[pl.ds(g * 16, 16)] = m
            return 0

        lax.fori_loop(0, _NBV, gbody, 0)

        def hbody(h, _):
            base = h * 256
            m = bmax[pl.ds(base, 16)]
            for j in range(1, 16):
                m = jnp.maximum(m, bmax[pl.ds(base + 16 * j, 16)])
            smax[pl.ds(h * 16, 16)] = m
            return 0

        lax.fori_loop(0, _NSV, hbody, 0)

        for h in range(_NSV):
            skey[pl.ds(h * 16, 16)] = _keyify(smax[pl.ds(h * 16, 16)])

        tA = _select_kth_key(skey, _NSV, _K)
        tAf = _unkey(tA)

        def sbody(g, ptr):
            v = bmax[pl.ds(g * 16, 16)]
            m = v >= tAf
            plsc.store_compressed(sv_val.at[pl.ds(ptr, 16)], v, mask=m)
            plsc.store_compressed(sv_id.at[pl.ds(ptr, 16)], g * 16 + iota, mask=m)
            return ptr + _count(m)

        n1 = lax.fori_loop(0, _NBV, sbody, jnp.int32(0))
        nv1 = (n1 + 15) // 16

        def kbody(j, _):
            sv_key[pl.ds(j * 16, 16)] = _keyify(sv_val[pl.ds(j * 16, 16)])
            return 0

        lax.fori_loop(0, nv1, kbody, 0)
        sv_key[pl.ds(n1, 16)] = jnp.zeros((16,), jnp.uint32)

        t1 = _select_kth_key(sv_key, nv1, _K)
        t1f = _unkey(t1)

        def s2body(j, ptr):
            v = sv_val[pl.ds(j * 16, 16)]
            ids = sv_id[pl.ds(j * 16, 16)]
            m = (v >= t1f) & ((j * 16 + iota) < n1)
            plsc.store_compressed(s2_id.at[pl.ds(ptr, 16)], ids, mask=m)
            return ptr + _count(m)

        n2 = lax.fori_loop(0, nv1, s2body, jnp.int32(0))
        s2_id[pl.ds(n2, 16)] = jnp.zeros((16,), jnp.int32)
        nb2 = (n2 + 15) // 16

        def cbody(j, ptr):
            ids = s2_id[pl.ds(j * 16, 16)]
            valid = (j * 16 + iota) < n2
            base = (ids >> 4) * 256 + (ids & 15)
            for jj in range(16):
                idxv = base + 16 * jj
                vals = plsc.load_gather(rowbuf, [idxv])
                m = (vals >= t1f) & valid
                plsc.store_compressed(c_val.at[pl.ds(ptr, 16)], vals, mask=m)
                plsc.store_compressed(c_idx.at[pl.ds(ptr, 16)], idxv, mask=m)
                ptr = jnp.minimum(ptr + _count(m), _CCAP)
            return ptr

        nc = lax.fori_loop(0, nb2, cbody, jnp.int32(0))
        nvc = (nc + 15) // 16

        def ckbody(j, _):
            c_key[pl.ds(j * 16, 16)] = _keyify(c_val[pl.ds(j * 16, 16)])
            return 0

        lax.fori_loop(0, nvc, ckbody, 0)
        c_key[pl.ds(nc, 16)] = jnp.zeros((16,), jnp.uint32)

        tkey = _select_kth_key(c_key, nvc, _K)

        def cgbody(j, cnt):
            return cnt + jnp.where(c_key[pl.ds(j * 16, 16)] > tkey, 1, 0)

        cgv = lax.fori_loop(0, nvc, cgbody, jnp.zeros((16,), jnp.int32))
        need = _K - jnp.sum(cgv)

        def eqbody(j, ptr):
            m = c_key[pl.ds(j * 16, 16)] == tkey
            plsc.store_compressed(eq_idx.at[pl.ds(ptr, 16)],
                                  c_idx[pl.ds(j * 16, 16)], mask=m)
            return ptr + _count(m)

        ne = lax.fori_loop(0, nvc, eqbody, jnp.int32(0))
        eq_idx[pl.ds(ne, 16)] = jnp.full((16,), jnp.int32(0x7FFFFFFF))
        nve = (ne + 15) // 16

        def lstep(i, L):
            cand = L | jnp.full((16,), jnp.int32(1) << (14 - i), jnp.int32)

            def lcnt(j, cnt):
                return cnt + jnp.where(eq_idx[pl.ds(j * 16, 16)] < cand, 1, 0)

            cnt = lax.fori_loop(0, nve, lcnt, jnp.zeros((16,), jnp.int32))
            return jnp.where(jnp.sum(cnt) <= need - 1, cand, L)

        L = lax.fori_loop(0, 15, lstep, jnp.zeros((16,), jnp.int32))

        rowbase = jnp.full((16,), row * _N, jnp.int32)

        def keepbody(j, ptr):
            kv = c_key[pl.ds(j * 16, 16)]
            iv = c_idx[pl.ds(j * 16, 16)]
            m = (kv > tkey) | ((kv == tkey) & (iv <= L))
            plsc.store_compressed(st_val.at[pl.ds(ptr, 16)],
                                  c_val[pl.ds(j * 16, 16)], mask=m)
            plsc.store_compressed(st_idx.at[pl.ds(ptr, 16)], iv + rowbase, mask=m)
            return ptr + _count(m)

        lax.fori_loop(0, nvc, keepbody, jnp.int32(0))

        for j in range(_K // 16):
            k_val[pl.ds(j * 16, 16)] = st_val[pl.ds(j * 16, 16)]
            k_idx[pl.ds(j * 16, 16)] = st_idx[pl.ds(j * 16, 16)]

        pltpu.sync_copy(zbuf, o_hbm.at[pl.ds(row * _N, _N)])
        pltpu.async_copy(k_val, o_hbm.at[k_idx], sem).wait()
        return 0

    lax.fori_loop(0, _RPW, do_row, 0)


_sc_call = functools.partial(
    pl.kernel,
    mesh=plsc.VectorSubcoreMesh(core_axis_name="c", subcore_axis_name="s"),
    compiler_params=pltpu.CompilerParams(needs_layout_passes=False),
    out_type=jax.ShapeDtypeStruct((_ROWS * _N,), jnp.float32),
    scratch_types=[
        pltpu.VMEM((_N,), jnp.float32),
        pltpu.VMEM((_N,), jnp.float32),
        pltpu.VMEM((_N // 16,), jnp.float32),
        pltpu.VMEM((_NBV,), jnp.float32),
        pltpu.VMEM((_NBV,), jnp.uint32),
        pltpu.VMEM((_SVCAP + 16,), jnp.float32),
        pltpu.VMEM((_SVCAP + 16,), jnp.int32),
        pltpu.VMEM((_SVCAP + 16,), jnp.uint32),
        pltpu.VMEM((_SVCAP + 16,), jnp.int32),
        pltpu.VMEM((_CCAP + 16,), jnp.float32),
        pltpu.VMEM((_CCAP + 16,), jnp.int32),
        pltpu.VMEM((_CCAP + 16,), jnp.uint32),
        pltpu.VMEM((_CCAP + 16,), jnp.int32),
        pltpu.VMEM((_K + 16,), jnp.float32),
        pltpu.VMEM((_K + 16,), jnp.int32),
        pltpu.VMEM((_K,), jnp.float32),
        pltpu.VMEM((_K,), jnp.int32),
        pltpu.SemaphoreType.DMA,
    ],
)(_body)


def kernel(x):
    out = _sc_call(x.reshape(-1))
    return out.reshape(x.shape)


# speedup vs baseline: 12.0347x; 1.4950x over previous
import functools

import jax
import jax.numpy as jnp
from jax import lax
from jax.experimental import pallas as pl
from jax.experimental.pallas import tpu as pltpu
from jax.experimental.pallas import tpu_sc as plsc

_K = 64
_N = 32768
_ROWS = 128
_NC = 2
_NS = 16
_NW = _NC * _NS
_RPW = _ROWS // _NW
_NBV = _N // 256
_NSV = _NBV // 16
_SVCAP = 2048
_CCAP = 4096


def _keyify(v):
    u = lax.bitcast_convert_type(v, jnp.uint32)
    return u ^ ((u >> jnp.uint32(31)) * jnp.uint32(0x7FFFFFFF)
                + jnp.uint32(0x80000000))


def _unkey(key):
    pos = key >> jnp.uint32(31)
    u = key ^ (jnp.uint32(0x80000000)
               + (jnp.uint32(1) - pos) * jnp.uint32(0x7FFFFFFF))
    return lax.bitcast_convert_type(u, jnp.float32)


def _count(m):
    return jnp.sum(jnp.where(m, 1, 0))


def _select_kth_key(key_ref, nv, k):
    def bit_step(i, t):
        sh = (jnp.uint32(31) - i.astype(jnp.uint32))
        cand = t | jnp.full((16,), jnp.uint32(1) << sh, jnp.uint32)

        def cbody(j, cnt):
            kv = key_ref[pl.ds(j * 16, 16)]
            return cnt + jnp.where(kv >= cand, 1, 0)

        cnt = lax.fori_loop(0, nv, cbody, jnp.zeros((16,), jnp.int32))
        return jnp.where(jnp.sum(cnt) >= k, cand, t)

    return lax.fori_loop(0, 32, bit_step, jnp.zeros((16,), jnp.uint32))


def _body(x_hbm, o_hbm, rowbuf, zbuf, bmax, smax, skey, sv_val, sv_id, sv_key,
          s2_id, c_val, c_idx, c_key, eq_idx, st_val, st_idx):
    wid = lax.axis_index("s") * _NC + lax.axis_index("c")
    iota = jnp.arange(16, dtype=jnp.int32)
    zero16f = jnp.zeros((16,), jnp.float32)

    def zinit(i, _):
        zbuf[pl.ds(i * 16, 16)] = zero16f
        return 0

    lax.fori_loop(0, _N // 16, zinit, 0)

    def do_row(r, _):
        row = wid * _RPW + r
        pltpu.sync_copy(x_hbm.at[row], rowbuf)

        def gbody(g, _):
            base = g * 256
            m = rowbuf[pl.ds(base, 16)]
            for j in range(1, 16):
                m = jnp.maximum(m, rowbuf[pl.ds(base + 16 * j, 16)])
            bmax[pl.ds(g * 16, 16)] = m
            return 0

        lax.fori_loop(0, _NBV, gbody, 0)

        def hbody(h, _):
            base = h * 256
            m = bmax[pl.ds(base, 16)]
            for j in range(1, 16):
                m = jnp.maximum(m, bmax[pl.ds(base + 16 * j, 16)])
            smax[pl.ds(h * 16, 16)] = m
            return 0

        lax.fori_loop(0, _NSV, hbody, 0)

        for h in range(_NSV):
            skey[pl.ds(h * 16, 16)] = _keyify(smax[pl.ds(h * 16, 16)])

        tA = _select_kth_key(skey, _NSV, _K)
        tAf = _unkey(tA)

        def sbody(g, ptr):
            v = bmax[pl.ds(g * 16, 16)]
            m = v >= tAf
            plsc.store_compressed(sv_val.at[pl.ds(ptr, 16)], v, mask=m)
            plsc.store_compressed(sv_id.at[pl.ds(ptr, 16)], g * 16 + iota, mask=m)
            return ptr + _count(m)

        n1 = lax.fori_loop(0, _NBV, sbody, jnp.int32(0))
        nv1 = (n1 + 15) // 16

        def kbody(j, _):
            sv_key[pl.ds(j * 16, 16)] = _keyify(sv_val[pl.ds(j * 16, 16)])
            return 0

        lax.fori_loop(0, nv1, kbody, 0)
        sv_key[pl.ds(n1, 16)] = jnp.zeros((16,), jnp.uint32)

        t1 = _select_kth_key(sv_key, nv1, _K)
        t1f = _unkey(t1)

        def s2body(j, ptr):
            v = sv_val[pl.ds(j * 16, 16)]
            ids = sv_id[pl.ds(j * 16, 16)]
            m = (v >= t1f) & ((j * 16 + iota) < n1)
            plsc.store_compressed(s2_id.at[pl.ds(ptr, 16)], ids, mask=m)
            return ptr + _count(m)

        n2 = lax.fori_loop(0, nv1, s2body, jnp.int32(0))
        s2_id[pl.ds(n2, 16)] = jnp.zeros((16,), jnp.int32)
        nb2 = (n2 + 15) // 16

        def cbody(j, ptr):
            ids = s2_id[pl.ds(j * 16, 16)]
            valid = (j * 16 + iota) < n2
            base = (ids >> 4) * 256 + (ids & 15)
            for jj in range(16):
                idxv = base + 16 * jj
                vals = plsc.load_gather(rowbuf, [idxv])
                m = (vals >= t1f) & valid
                plsc.store_compressed(c_val.at[pl.ds(ptr, 16)], vals, mask=m)
                plsc.store_compressed(c_idx.at[pl.ds(ptr, 16)], idxv, mask=m)
                ptr = jnp.minimum(ptr + _count(m), _CCAP)
            return ptr

        nc = lax.fori_loop(0, nb2, cbody, jnp.int32(0))
        nvc = (nc + 15) // 16

        def ckbody(j, _):
            c_key[pl.ds(j * 16, 16)] = _keyify(c_val[pl.ds(j * 16, 16)])
            return 0

        lax.fori_loop(0, nvc, ckbody, 0)
        c_key[pl.ds(nc, 16)] = jnp.zeros((16,), jnp.uint32)

        tkey = _select_kth_key(c_key, nvc, _K)

        def cgbody(j, cnt):
            return cnt + jnp.where(c_key[pl.ds(j * 16, 16)] > tkey, 1, 0)

        cgv = lax.fori_loop(0, nvc, cgbody, jnp.zeros((16,), jnp.int32))
        need = _K - jnp.sum(cgv)

        def eqbody(j, ptr):
            m = c_key[pl.ds(j * 16, 16)] == tkey
            plsc.store_compressed(eq_idx.at[pl.ds(ptr, 16)],
                                  c_idx[pl.ds(j * 16, 16)], mask=m)
            return ptr + _count(m)

        ne = lax.fori_loop(0, nvc, eqbody, jnp.int32(0))
        eq_idx[pl.ds(ne, 16)] = jnp.full((16,), jnp.int32(0x7FFFFFFF))
        nve = (ne + 15) // 16

        def lstep(i, L):
            cand = L | jnp.full((16,), jnp.int32(1) << (14 - i), jnp.int32)

            def lcnt(j, cnt):
                return cnt + jnp.where(eq_idx[pl.ds(j * 16, 16)] < cand, 1, 0)

            cnt = lax.fori_loop(0, nve, lcnt, jnp.zeros((16,), jnp.int32))
            return jnp.where(jnp.sum(cnt) <= need - 1, cand, L)

        L = lax.fori_loop(0, 15, lstep, jnp.zeros((16,), jnp.int32))

        def keepbody(j, ptr):
            kv = c_key[pl.ds(j * 16, 16)]
            iv = c_idx[pl.ds(j * 16, 16)]
            m = (kv > tkey) | ((kv == tkey) & (iv <= L))
            plsc.store_compressed(st_val.at[pl.ds(ptr, 16)],
                                  c_val[pl.ds(j * 16, 16)], mask=m)
            plsc.store_compressed(st_idx.at[pl.ds(ptr, 16)], iv, mask=m)
            return ptr + _count(m)

        lax.fori_loop(0, nvc, keepbody, jnp.int32(0))

        for j in range(_K // 16):
            plsc.store_scatter(zbuf, [st_idx[pl.ds(j * 16, 16)]],
                               st_val[pl.ds(j * 16, 16)])
        pltpu.sync_copy(zbuf, o_hbm.at[row])
        for j in range(_K // 16):
            plsc.store_scatter(zbuf, [st_idx[pl.ds(j * 16, 16)]], zero16f)
        return 0

    lax.fori_loop(0, _RPW, do_row, 0)


_sc_call = functools.partial(
    pl.kernel,
    mesh=plsc.VectorSubcoreMesh(core_axis_name="c", subcore_axis_name="s"),
    compiler_params=pltpu.CompilerParams(needs_layout_passes=False),
    out_type=jax.ShapeDtypeStruct((_ROWS, _N), jnp.float32),
    scratch_types=[
        pltpu.VMEM((_N,), jnp.float32),
        pltpu.VMEM((_N,), jnp.float32),
        pltpu.VMEM((_N // 16,), jnp.float32),
        pltpu.VMEM((_NBV,), jnp.float32),
        pltpu.VMEM((_NBV,), jnp.uint32),
        pltpu.VMEM((_SVCAP + 16,), jnp.float32),
        pltpu.VMEM((_SVCAP + 16,), jnp.int32),
        pltpu.VMEM((_SVCAP + 16,), jnp.uint32),
        pltpu.VMEM((_SVCAP + 16,), jnp.int32),
        pltpu.VMEM((_CCAP + 16,), jnp.float32),
        pltpu.VMEM((_CCAP + 16,), jnp.int32),
        pltpu.VMEM((_CCAP + 16,), jnp.uint32),
        pltpu.VMEM((_CCAP + 16,), jnp.int32),
        pltpu.VMEM((_K + 16,), jnp.float32),
        pltpu.VMEM((_K + 16,), jnp.int32),
    ],
)(_body)


def kernel(x):
    return _sc_call(x)


# speedup vs baseline: 13.7587x; 1.1433x over previous
import functools

import jax
import jax.numpy as jnp
from jax import lax
from jax.experimental import pallas as pl
from jax.experimental.pallas import tpu as pltpu
from jax.experimental.pallas import tpu_sc as plsc

_K = 64
_N = 32768
_ROWS = 128
_NC = 2
_NS = 16
_NW = _NC * _NS
_RPW = _ROWS // _NW
_NBV = _N // 256
_NSV = _NBV // 16
_SVCAP = 2048
_CCAP = 4096


def _keyify(v):
    u = lax.bitcast_convert_type(v, jnp.uint32)
    return u ^ ((u >> jnp.uint32(31)) * jnp.uint32(0x7FFFFFFF)
                + jnp.uint32(0x80000000))


def _unkey(key):
    pos = key >> jnp.uint32(31)
    u = key ^ (jnp.uint32(0x80000000)
               + (jnp.uint32(1) - pos) * jnp.uint32(0x7FFFFFFF))
    return lax.bitcast_convert_type(u, jnp.float32)


def _count(m):
    return jnp.sum(jnp.where(m, 1, 0))


def _select_kth_key(key_ref, nv, k):
    def bit_step(i, t):
        sh = (jnp.uint32(31) - i.astype(jnp.uint32))
        cand = t | jnp.full((16,), jnp.uint32(1) << sh, jnp.uint32)

        def cbody(j, cnt):
            kv = key_ref[pl.ds(j * 16, 16)]
            return cnt + jnp.where(kv >= cand, 1, 0)

        cnt = lax.fori_loop(0, nv, cbody, jnp.zeros((16,), jnp.int32))
        return jnp.where(jnp.sum(cnt) >= k, cand, t)

    return lax.fori_loop(0, 32, bit_step, jnp.zeros((16,), jnp.uint32))


def _body(x_hbm, o_hbm, rowbuf, zbuf, bmax, smax, skey, sv_val, sv_id, sv_key,
          s2_id, c_val, c_idx, c_key, eq_idx, st_val, st_idx):
    wid = lax.axis_index("s") * _NC + lax.axis_index("c")
    iota = jnp.arange(16, dtype=jnp.int32)
    zero16f = jnp.zeros((16,), jnp.float32)

    @plsc.parallel_loop(0, _N // 16, unroll=8)
    def _(i):
        zbuf[pl.ds(i * 16, 16)] = zero16f

    def do_row(r, _):
        row = wid * _RPW + r
        pltpu.sync_copy(x_hbm.at[row], rowbuf)

        @plsc.parallel_loop(0, _NBV, unroll=4)
        def _(g):
            base = g * 256
            m = rowbuf[pl.ds(base, 16)]
            for j in range(1, 16):
                m = jnp.maximum(m, rowbuf[pl.ds(base + 16 * j, 16)])
            bmax[pl.ds(g * 16, 16)] = m

        @plsc.parallel_loop(0, _NSV, unroll=2)
        def _(h):
            base = h * 256
            m = bmax[pl.ds(base, 16)]
            for j in range(1, 16):
                m = jnp.maximum(m, bmax[pl.ds(base + 16 * j, 16)])
            smax[pl.ds(h * 16, 16)] = m

        for h in range(_NSV):
            skey[pl.ds(h * 16, 16)] = _keyify(smax[pl.ds(h * 16, 16)])

        tA = _select_kth_key(skey, _NSV, _K)
        tAf = _unkey(tA)

        def sbody(g, ptr):
            v = bmax[pl.ds(g * 16, 16)]
            m = v >= tAf
            plsc.store_compressed(sv_val.at[pl.ds(ptr, 16)], v, mask=m)
            plsc.store_compressed(sv_id.at[pl.ds(ptr, 16)], g * 16 + iota, mask=m)
            return ptr + _count(m)

        n1 = lax.fori_loop(0, _NBV, sbody, jnp.int32(0))
        nv1 = (n1 + 15) // 16

        def kbody(j, _):
            sv_key[pl.ds(j * 16, 16)] = _keyify(sv_val[pl.ds(j * 16, 16)])
            return 0

        lax.fori_loop(0, nv1, kbody, 0)
        sv_key[pl.ds(n1, 16)] = jnp.zeros((16,), jnp.uint32)

        t1 = _select_kth_key(sv_key, nv1, _K)
        t1f = _unkey(t1)

        def s2body(j, ptr):
            v = sv_val[pl.ds(j * 16, 16)]
            ids = sv_id[pl.ds(j * 16, 16)]
            m = (v >= t1f) & ((j * 16 + iota) < n1)
            plsc.store_compressed(s2_id.at[pl.ds(ptr, 16)], ids, mask=m)
            return ptr + _count(m)

        n2 = lax.fori_loop(0, nv1, s2body, jnp.int32(0))
        s2_id[pl.ds(n2, 16)] = jnp.zeros((16,), jnp.int32)
        nb2 = (n2 + 15) // 16

        def cbody(j, ptr):
            ids = s2_id[pl.ds(j * 16, 16)]
            valid = (j * 16 + iota) < n2
            base = (ids >> 4) * 256 + (ids & 15)
            for jj in range(16):
                idxv = base + 16 * jj
                vals = plsc.load_gather(rowbuf, [idxv])
                m = (vals >= t1f) & valid
                plsc.store_compressed(c_val.at[pl.ds(ptr, 16)], vals, mask=m)
                plsc.store_compressed(c_idx.at[pl.ds(ptr, 16)], idxv, mask=m)
                ptr = jnp.minimum(ptr + _count(m), _CCAP)
            return ptr

        nc = lax.fori_loop(0, nb2, cbody, jnp.int32(0))
        nvc = (nc + 15) // 16

        def ckbody(j, _):
            c_key[pl.ds(j * 16, 16)] = _keyify(c_val[pl.ds(j * 16, 16)])
            return 0

        lax.fori_loop(0, nvc, ckbody, 0)
        c_key[pl.ds(nc, 16)] = jnp.zeros((16,), jnp.uint32)

        tkey = _select_kth_key(c_key, nvc, _K)

        def cgbody(j, cnt):
            return cnt + jnp.where(c_key[pl.ds(j * 16, 16)] > tkey, 1, 0)

        cgv = lax.fori_loop(0, nvc, cgbody, jnp.zeros((16,), jnp.int32))
        need = _K - jnp.sum(cgv)

        def eqbody(j, ptr):
            m = c_key[pl.ds(j * 16, 16)] == tkey
            plsc.store_compressed(eq_idx.at[pl.ds(ptr, 16)],
                                  c_idx[pl.ds(j * 16, 16)], mask=m)
            return ptr + _count(m)

        ne = lax.fori_loop(0, nvc, eqbody, jnp.int32(0))
        eq_idx[pl.ds(ne, 16)] = jnp.full((16,), jnp.int32(0x7FFFFFFF))
        nve = (ne + 15) // 16

        def lstep(i, L):
            cand = L | jnp.full((16,), jnp.int32(1) << (14 - i), jnp.int32)

            def lcnt(j, cnt):
                return cnt + jnp.where(eq_idx[pl.ds(j * 16, 16)] < cand, 1, 0)

            cnt = lax.fori_loop(0, nve, lcnt, jnp.zeros((16,), jnp.int32))
            return jnp.where(jnp.sum(cnt) <= need - 1, cand, L)

        L = lax.fori_loop(0, 15, lstep, jnp.zeros((16,), jnp.int32))

        def keepbody(j, ptr):
            kv = c_key[pl.ds(j * 16, 16)]
            iv = c_idx[pl.ds(j * 16, 16)]
            m = (kv > tkey) | ((kv == tkey) & (iv <= L))
            plsc.store_compressed(st_val.at[pl.ds(ptr, 16)],
                                  c_val[pl.ds(j * 16, 16)], mask=m)
            plsc.store_compressed(st_idx.at[pl.ds(ptr, 16)], iv, mask=m)
            return ptr + _count(m)

        lax.fori_loop(0, nvc, keepbody, jnp.int32(0))

        for j in range(_K // 16):
            plsc.store_scatter(zbuf, [st_idx[pl.ds(j * 16, 16)]],
                               st_val[pl.ds(j * 16, 16)])
        pltpu.sync_copy(zbuf, o_hbm.at[row])
        for j in range(_K // 16):
            plsc.store_scatter(zbuf, [st_idx[pl.ds(j * 16, 16)]], zero16f)
        return 0

    lax.fori_loop(0, _RPW, do_row, 0)


_sc_call = functools.partial(
    pl.kernel,
    mesh=plsc.VectorSubcoreMesh(core_axis_name="c", subcore_axis_name="s"),
    compiler_params=pltpu.CompilerParams(needs_layout_passes=False),
    out_type=jax.ShapeDtypeStruct((_ROWS, _N), jnp.float32),
    scratch_types=[
        pltpu.VMEM((_N,), jnp.float32),
        pltpu.VMEM((_N,), jnp.float32),
        pltpu.VMEM((_N // 16,), jnp.float32),
        pltpu.VMEM((_NBV,), jnp.float32),
        pltpu.VMEM((_NBV,), jnp.uint32),
        pltpu.VMEM((_SVCAP + 16,), jnp.float32),
        pltpu.VMEM((_SVCAP + 16,), jnp.int32),
        pltpu.VMEM((_SVCAP + 16,), jnp.uint32),
        pltpu.VMEM((_SVCAP + 16,), jnp.int32),
        pltpu.VMEM((_CCAP + 16,), jnp.float32),
        pltpu.VMEM((_CCAP + 16,), jnp.int32),
        pltpu.VMEM((_CCAP + 16,), jnp.uint32),
        pltpu.VMEM((_CCAP + 16,), jnp.int32),
        pltpu.VMEM((_K + 16,), jnp.float32),
        pltpu.VMEM((_K + 16,), jnp.int32),
    ],
)(_body)


def kernel(x):
    return _sc_call(x)


# speedup vs baseline: 14.9105x; 1.0837x over previous
import functools

import jax
import jax.numpy as jnp
from jax import lax
from jax.experimental import pallas as pl
from jax.experimental.pallas import tpu as pltpu
from jax.experimental.pallas import tpu_sc as plsc

_K = 64
_N = 32768
_ROWS = 128
_NC = 2
_NS = 16
_NW = _NC * _NS
_RPW = _ROWS // _NW
_NBV = _N // 256
_NSV = _NBV // 16
_SVCAP = 2048
_CCAP = 4096


def _keyify(v):
    u = lax.bitcast_convert_type(v, jnp.uint32)
    return u ^ ((u >> jnp.uint32(31)) * jnp.uint32(0x7FFFFFFF)
                + jnp.uint32(0x80000000))


def _unkey(key):
    pos = key >> jnp.uint32(31)
    u = key ^ (jnp.uint32(0x80000000)
               + (jnp.uint32(1) - pos) * jnp.uint32(0x7FFFFFFF))
    return lax.bitcast_convert_type(u, jnp.float32)


def _count(m):
    return jnp.sum(jnp.where(m, 1, 0))


def _select_kth_key(key_ref, nv, k):
    def bit_step(i, t):
        sh = (jnp.uint32(31) - i.astype(jnp.uint32))
        cand = t | jnp.full((16,), jnp.uint32(1) << sh, jnp.uint32)

        @plsc.parallel_loop(0, nv, unroll=4, carry=jnp.zeros((16,), jnp.int32))
        def cnt(j, acc):
            kv = key_ref[pl.ds(j * 16, 16)]
            return acc + jnp.where(kv >= cand, 1, 0)

        return jnp.where(jnp.sum(cnt) >= k, cand, t)

    return lax.fori_loop(0, 32, bit_step, jnp.zeros((16,), jnp.uint32))


def _body(x_hbm, o_hbm, rowbuf, zbuf, bmax, smax, skey, sv_val, sv_id, sv_key,
          s2_id, c_val, c_idx, c_key, eq_idx, st_val, st_idx):
    wid = lax.axis_index("s") * _NC + lax.axis_index("c")
    iota = jnp.arange(16, dtype=jnp.int32)
    zero16f = jnp.zeros((16,), jnp.float32)

    @plsc.parallel_loop(0, _N // 16, unroll=8)
    def _(i):
        zbuf[pl.ds(i * 16, 16)] = zero16f

    def do_row(r, _):
        row = wid * _RPW + r
        pltpu.sync_copy(x_hbm.at[row], rowbuf)

        @plsc.parallel_loop(0, _NBV, unroll=4)
        def _(g):
            base = g * 256
            m = rowbuf[pl.ds(base, 16)]
            for j in range(1, 16):
                m = jnp.maximum(m, rowbuf[pl.ds(base + 16 * j, 16)])
            bmax[pl.ds(g * 16, 16)] = m

        @plsc.parallel_loop(0, _NSV, unroll=2)
        def _(h):
            base = h * 256
            m = bmax[pl.ds(base, 16)]
            for j in range(1, 16):
                m = jnp.maximum(m, bmax[pl.ds(base + 16 * j, 16)])
            smax[pl.ds(h * 16, 16)] = m

        for h in range(_NSV):
            skey[pl.ds(h * 16, 16)] = _keyify(smax[pl.ds(h * 16, 16)])

        tA = _select_kth_key(skey, _NSV, _K)
        tAf = _unkey(tA)

        def sbody(gg, ptr):
            vs, ms, cs = [], [], []
            for u in range(8):
                v = bmax[pl.ds((gg * 8 + u) * 16, 16)]
                m = v >= tAf
                vs.append(v)
                ms.append(m)
                cs.append(_count(m))
            for u in range(8):
                plsc.store_compressed(sv_val.at[pl.ds(ptr, 16)], vs[u],
                                      mask=ms[u])
                plsc.store_compressed(sv_id.at[pl.ds(ptr, 16)],
                                      (gg * 8 + u) * 16 + iota, mask=ms[u])
                ptr = ptr + cs[u]
            return ptr

        n1 = lax.fori_loop(0, _NBV // 8, sbody, jnp.int32(0))
        nv1 = (n1 + 15) // 16

        def kbody(j, _):
            sv_key[pl.ds(j * 16, 16)] = _keyify(sv_val[pl.ds(j * 16, 16)])
            return 0

        lax.fori_loop(0, nv1, kbody, 0)
        sv_key[pl.ds(n1, 16)] = jnp.zeros((16,), jnp.uint32)

        t1 = _select_kth_key(sv_key, nv1, _K)
        t1f = _unkey(t1)

        def s2body(j, ptr):
            v = sv_val[pl.ds(j * 16, 16)]
            ids = sv_id[pl.ds(j * 16, 16)]
            m = (v >= t1f) & ((j * 16 + iota) < n1)
            plsc.store_compressed(s2_id.at[pl.ds(ptr, 16)], ids, mask=m)
            return ptr + _count(m)

        n2 = lax.fori_loop(0, nv1, s2body, jnp.int32(0))
        s2_id[pl.ds(n2, 16)] = jnp.zeros((16,), jnp.int32)
        nb2 = (n2 + 15) // 16

        def cbody(j, ptr):
            ids = s2_id[pl.ds(j * 16, 16)]
            valid = (j * 16 + iota) < n2
            base = (ids >> 4) * 256 + (ids & 15)
            gv, gi, ms, cs = [], [], [], []
            for jj in range(16):
                idxv = base + 16 * jj
                vals = plsc.load_gather(rowbuf, [idxv])
                m = (vals >= t1f) & valid
                gv.append(vals)
                gi.append(idxv)
                ms.append(m)
                cs.append(_count(m))
            for jj in range(16):
                plsc.store_compressed(c_val.at[pl.ds(ptr, 16)], gv[jj],
                                      mask=ms[jj])
                plsc.store_compressed(c_idx.at[pl.ds(ptr, 16)], gi[jj],
                                      mask=ms[jj])
                ptr = jnp.minimum(ptr + cs[jj], _CCAP)
            return ptr

        nc = lax.fori_loop(0, nb2, cbody, jnp.int32(0))
        nvc = (nc + 15) // 16

        def ckbody(j, _):
            c_key[pl.ds(j * 16, 16)] = _keyify(c_val[pl.ds(j * 16, 16)])
            return 0

        lax.fori_loop(0, nvc, ckbody, 0)
        c_key[pl.ds(nc, 16)] = jnp.zeros((16,), jnp.uint32)

        tkey = _select_kth_key(c_key, nvc, _K)

        def cgbody(j, cnt):
            return cnt + jnp.where(c_key[pl.ds(j * 16, 16)] > tkey, 1, 0)

        cgv = lax.fori_loop(0, nvc, cgbody, jnp.zeros((16,), jnp.int32))
        need = _K - jnp.sum(cgv)

        def eqbody(j, ptr):
            m = c_key[pl.ds(j * 16, 16)] == tkey
            plsc.store_compressed(eq_idx.at[pl.ds(ptr, 16)],
                                  c_idx[pl.ds(j * 16, 16)], mask=m)
            return ptr + _count(m)

        ne = lax.fori_loop(0, nvc, eqbody, jnp.int32(0))
        eq_idx[pl.ds(ne, 16)] = jnp.full((16,), jnp.int32(0x7FFFFFFF))
        nve = (ne + 15) // 16

        def lstep(i, L):
            cand = L | jnp.full((16,), jnp.int32(1) << (14 - i), jnp.int32)

            def lcnt(j, cnt):
                return cnt + jnp.where(eq_idx[pl.ds(j * 16, 16)] < cand, 1, 0)

            cnt = lax.fori_loop(0, nve, lcnt, jnp.zeros((16,), jnp.int32))
            return jnp.where(jnp.sum(cnt) <= need - 1, cand, L)

        L = lax.fori_loop(0, 15, lstep, jnp.zeros((16,), jnp.int32))

        def keepbody(j, ptr):
            kv = c_key[pl.ds(j * 16, 16)]
            iv = c_idx[pl.ds(j * 16, 16)]
            m = (kv > tkey) | ((kv == tkey) & (iv <= L))
            plsc.store_compressed(st_val.at[pl.ds(ptr, 16)],
                                  c_val[pl.ds(j * 16, 16)], mask=m)
            plsc.store_compressed(st_idx.at[pl.ds(ptr, 16)], iv, mask=m)
            return ptr + _count(m)

        lax.fori_loop(0, nvc, keepbody, jnp.int32(0))

        for j in range(_K // 16):
            plsc.store_scatter(zbuf, [st_idx[pl.ds(j * 16, 16)]],
                               st_val[pl.ds(j * 16, 16)])
        pltpu.sync_copy(zbuf, o_hbm.at[row])
        for j in range(_K // 16):
            plsc.store_scatter(zbuf, [st_idx[pl.ds(j * 16, 16)]], zero16f)
        return 0

    lax.fori_loop(0, _RPW, do_row, 0)


_sc_call = functools.partial(
    pl.kernel,
    mesh=plsc.VectorSubcoreMesh(core_axis_name="c", subcore_axis_name="s"),
    compiler_params=pltpu.CompilerParams(needs_layout_passes=False),
    out_type=jax.ShapeDtypeStruct((_ROWS, _N), jnp.float32),
    scratch_types=[
        pltpu.VMEM((_N,), jnp.float32),
        pltpu.VMEM((_N,), jnp.float32),
        pltpu.VMEM((_N // 16,), jnp.float32),
        pltpu.VMEM((_NBV,), jnp.float32),
        pltpu.VMEM((_NBV,), jnp.uint32),
        pltpu.VMEM((_SVCAP + 16,), jnp.float32),
        pltpu.VMEM((_SVCAP + 16,), jnp.int32),
        pltpu.VMEM((_SVCAP + 16,), jnp.uint32),
        pltpu.VMEM((_SVCAP + 16,), jnp.int32),
        pltpu.VMEM((_CCAP + 16,), jnp.float32),
        pltpu.VMEM((_CCAP + 16,), jnp.int32),
        pltpu.VMEM((_CCAP + 16,), jnp.uint32),
        pltpu.VMEM((_CCAP + 16,), jnp.int32),
        pltpu.VMEM((_K + 16,), jnp.float32),
        pltpu.VMEM((_K + 16,), jnp.int32),
    ],
)(_body)


def kernel(x):
    return _sc_call(x)


# speedup vs baseline: 15.7873x; 1.0588x over previous
import functools

import jax
import jax.numpy as jnp
from jax import lax
from jax.experimental import pallas as pl
from jax.experimental.pallas import tpu as pltpu
from jax.experimental.pallas import tpu_sc as plsc

_K = 64
_N = 32768
_ROWS = 128
_NC = 2
_NS = 16
_NW = _NC * _NS
_RPW = _ROWS // _NW
_NBV = _N // 256
_NSV = _NBV // 16
_SVCAP = 2048
_CCAP = 4096


def _keyify(v):
    u = lax.bitcast_convert_type(v, jnp.uint32)
    return u ^ ((u >> jnp.uint32(31)) * jnp.uint32(0x7FFFFFFF)
                + jnp.uint32(0x80000000))


def _unkey(key):
    pos = key >> jnp.uint32(31)
    u = key ^ (jnp.uint32(0x80000000)
               + (jnp.uint32(1) - pos) * jnp.uint32(0x7FFFFFFF))
    return lax.bitcast_convert_type(u, jnp.float32)


def _count(m):
    return jnp.sum(jnp.where(m, 1, 0))


def _select_kth_key(key_ref, nv, k):
    def bit_step(i, t):
        sh = (jnp.uint32(31) - i.astype(jnp.uint32))
        cand = t | jnp.full((16,), jnp.uint32(1) << sh, jnp.uint32)

        @plsc.parallel_loop(0, nv, unroll=4, carry=jnp.zeros((16,), jnp.int32))
        def cnt(j, acc):
            kv = key_ref[pl.ds(j * 16, 16)]
            return acc + jnp.where(kv >= cand, 1, 0)

        return jnp.where(jnp.sum(cnt) >= k, cand, t)

    return lax.fori_loop(0, 32, bit_step, jnp.zeros((16,), jnp.uint32))


def _body(x_hbm, o_hbm, rowbuf, rowbuf2, zbuf, bmax, smax, skey, sv_val,
          sv_id, sv_key, s2_id, c_val, c_idx, c_key, eq_idx, st_val, st_idx,
          st_val2, st_idx2, isem, osem):
    wid = lax.axis_index("s") * _NC + lax.axis_index("c")
    iota = jnp.arange(16, dtype=jnp.int32)
    zero16f = jnp.zeros((16,), jnp.float32)

    @plsc.parallel_loop(0, _N // 16, unroll=8)
    def _(i):
        zbuf[pl.ds(i * 16, 16)] = zero16f

    def select_row(row, rbuf, stv, sti):
        @plsc.parallel_loop(0, _NBV, unroll=4)
        def _(g):
            base = g * 256
            m = rbuf[pl.ds(base, 16)]
            for j in range(1, 16):
                m = jnp.maximum(m, rbuf[pl.ds(base + 16 * j, 16)])
            bmax[pl.ds(g * 16, 16)] = m

        @plsc.parallel_loop(0, _NSV, unroll=2)
        def _(h):
            base = h * 256
            m = bmax[pl.ds(base, 16)]
            for j in range(1, 16):
                m = jnp.maximum(m, bmax[pl.ds(base + 16 * j, 16)])
            smax[pl.ds(h * 16, 16)] = m

        for h in range(_NSV):
            skey[pl.ds(h * 16, 16)] = _keyify(smax[pl.ds(h * 16, 16)])

        tA = _select_kth_key(skey, _NSV, _K)
        tAf = _unkey(tA)

        def sbody(gg, ptr):
            vs, ms, cs = [], [], []
            for u in range(8):
                v = bmax[pl.ds((gg * 8 + u) * 16, 16)]
                m = v >= tAf
                vs.append(v)
                ms.append(m)
                cs.append(_count(m))
            for u in range(8):
                plsc.store_compressed(sv_val.at[pl.ds(ptr, 16)], vs[u],
                                      mask=ms[u])
                plsc.store_compressed(sv_id.at[pl.ds(ptr, 16)],
                                      (gg * 8 + u) * 16 + iota, mask=ms[u])
                ptr = ptr + cs[u]
            return ptr

        n1 = lax.fori_loop(0, _NBV // 8, sbody, jnp.int32(0))
        nv1 = (n1 + 15) // 16

        def kbody(j, _):
            sv_key[pl.ds(j * 16, 16)] = _keyify(sv_val[pl.ds(j * 16, 16)])
            return 0

        lax.fori_loop(0, nv1, kbody, 0)
        sv_key[pl.ds(n1, 16)] = jnp.zeros((16,), jnp.uint32)

        t1 = _select_kth_key(sv_key, nv1, _K)
        t1f = _unkey(t1)

        def s2body(j, ptr):
            v = sv_val[pl.ds(j * 16, 16)]
            ids = sv_id[pl.ds(j * 16, 16)]
            m = (v >= t1f) & ((j * 16 + iota) < n1)
            plsc.store_compressed(s2_id.at[pl.ds(ptr, 16)], ids, mask=m)
            return ptr + _count(m)

        n2 = lax.fori_loop(0, nv1, s2body, jnp.int32(0))
        s2_id[pl.ds(n2, 16)] = jnp.zeros((16,), jnp.int32)
        nb2 = (n2 + 15) // 16

        def cbody(j, ptr):
            ids = s2_id[pl.ds(j * 16, 16)]
            valid = (j * 16 + iota) < n2
            base = (ids >> 4) * 256 + (ids & 15)
            gv, gi, ms, cs = [], [], [], []
            for jj in range(16):
                idxv = base + 16 * jj
                vals = plsc.load_gather(rbuf, [idxv])
                m = (vals >= t1f) & valid
                gv.append(vals)
                gi.append(idxv)
                ms.append(m)
                cs.append(_count(m))
            for jj in range(16):
                plsc.store_compressed(c_val.at[pl.ds(ptr, 16)], gv[jj],
                                      mask=ms[jj])
                plsc.store_compressed(c_idx.at[pl.ds(ptr, 16)], gi[jj],
                                      mask=ms[jj])
                ptr = jnp.minimum(ptr + cs[jj], _CCAP)
            return ptr

        nc = lax.fori_loop(0, nb2, cbody, jnp.int32(0))
        nvc = (nc + 15) // 16

        def ckbody(j, _):
            c_key[pl.ds(j * 16, 16)] = _keyify(c_val[pl.ds(j * 16, 16)])
            return 0

        lax.fori_loop(0, nvc, ckbody, 0)
        c_key[pl.ds(nc, 16)] = jnp.zeros((16,), jnp.uint32)

        tkey = _select_kth_key(c_key, nvc, _K)

        def cgbody(j, cnt):
            return cnt + jnp.where(c_key[pl.ds(j * 16, 16)] > tkey, 1, 0)

        cgv = lax.fori_loop(0, nvc, cgbody, jnp.zeros((16,), jnp.int32))
        need = _K - jnp.sum(cgv)

        def eqbody(j, ptr):
            m = c_key[pl.ds(j * 16, 16)] == tkey
            plsc.store_compressed(eq_idx.at[pl.ds(ptr, 16)],
                                  c_idx[pl.ds(j * 16, 16)], mask=m)
            return ptr + _count(m)

        ne = lax.fori_loop(0, nvc, eqbody, jnp.int32(0))
        eq_idx[pl.ds(ne, 16)] = jnp.full((16,), jnp.int32(0x7FFFFFFF))
        nve = (ne + 15) // 16

        def lstep(i, L):
            cand = L | jnp.full((16,), jnp.int32(1) << (14 - i), jnp.int32)

            def lcnt(j, cnt):
                return cnt + jnp.where(eq_idx[pl.ds(j * 16, 16)] < cand, 1, 0)

            cnt = lax.fori_loop(0, nve, lcnt, jnp.zeros((16,), jnp.int32))
            return jnp.where(jnp.sum(cnt) <= need - 1, cand, L)

        L = lax.fori_loop(0, 15, lstep, jnp.zeros((16,), jnp.int32))

        def keepbody(j, ptr):
            kv = c_key[pl.ds(j * 16, 16)]
            iv = c_idx[pl.ds(j * 16, 16)]
            m = (kv > tkey) | ((kv == tkey) & (iv <= L))
            plsc.store_compressed(stv.at[pl.ds(ptr, 16)],
                                  c_val[pl.ds(j * 16, 16)], mask=m)
            plsc.store_compressed(sti.at[pl.ds(ptr, 16)], iv, mask=m)
            return ptr + _count(m)

        lax.fori_loop(0, nvc, keepbody, jnp.int32(0))

    row0 = wid * _RPW
    bufs = [rowbuf, rowbuf2]
    sts = [(st_val, st_idx), (st_val2, st_idx2)]
    ih = pltpu.async_copy(x_hbm.at[row0], bufs[0], isem)
    oh = None
    for r in range(_RPW):
        ih.wait()
        if r + 1 < _RPW:
            ih = pltpu.async_copy(x_hbm.at[row0 + r + 1], bufs[(r + 1) % 2],
                                  isem)
        stv, sti = sts[r % 2]
        select_row(row0 + r, bufs[r % 2], stv, sti)
        if oh is not None:
            oh.wait()
            _, psti = sts[(r + 1) % 2]
            for j in range(_K // 16):
                plsc.store_scatter(zbuf, [psti[pl.ds(j * 16, 16)]], zero16f)
        for j in range(_K // 16):
            plsc.store_scatter(zbuf, [sti[pl.ds(j * 16, 16)]],
                               stv[pl.ds(j * 16, 16)])
        oh = pltpu.async_copy(zbuf, o_hbm.at[row0 + r], osem)
    oh.wait()


_sc_call = functools.partial(
    pl.kernel,
    mesh=plsc.VectorSubcoreMesh(core_axis_name="c", subcore_axis_name="s"),
    compiler_params=pltpu.CompilerParams(needs_layout_passes=False),
    out_type=jax.ShapeDtypeStruct((_ROWS, _N), jnp.float32),
    scratch_types=[
        pltpu.VMEM((_N,), jnp.float32),
        pltpu.VMEM((_N,), jnp.float32),
        pltpu.VMEM((_N,), jnp.float32),
        pltpu.VMEM((_N // 16,), jnp.float32),
        pltpu.VMEM((_NBV,), jnp.float32),
        pltpu.VMEM((_NBV,), jnp.uint32),
        pltpu.VMEM((_SVCAP + 16,), jnp.float32),
        pltpu.VMEM((_SVCAP + 16,), jnp.int32),
        pltpu.VMEM((_SVCAP + 16,), jnp.uint32),
        pltpu.VMEM((_SVCAP + 16,), jnp.int32),
        pltpu.VMEM((_CCAP + 16,), jnp.float32),
        pltpu.VMEM((_CCAP + 16,), jnp.int32),
        pltpu.VMEM((_CCAP + 16,), jnp.uint32),
        pltpu.VMEM((_CCAP + 16,), jnp.int32),
        pltpu.VMEM((_K + 16,), jnp.float32),
        pltpu.VMEM((_K + 16,), jnp.int32),
        pltpu.VMEM((_K + 16,), jnp.float32),
        pltpu.VMEM((_K + 16,), jnp.int32),
        pltpu.SemaphoreType.DMA,
        pltpu.SemaphoreType.DMA,
    ],
)(_body)


def kernel(x):
    return _sc_call(x)


# speedup vs baseline: 16.9644x; 1.0746x over previous
import functools

import jax
import jax.numpy as jnp
from jax import lax
from jax.experimental import pallas as pl
from jax.experimental.pallas import tpu as pltpu
from jax.experimental.pallas import tpu_sc as plsc

_K = 64
_N = 32768
_ROWS = 128
_NC = 2
_NS = 16
_NW = _NC * _NS
_RPW = _ROWS // _NW
_NBV = _N // 256
_NSV = _NBV // 16
_SVCAP = 2048
_CCAP = 4096


def _keyify(v):
    u = lax.bitcast_convert_type(v, jnp.uint32)
    return u ^ ((u >> jnp.uint32(31)) * jnp.uint32(0x7FFFFFFF)
                + jnp.uint32(0x80000000))


def _unkey(key):
    pos = key >> jnp.uint32(31)
    u = key ^ (jnp.uint32(0x80000000)
               + (jnp.uint32(1) - pos) * jnp.uint32(0x7FFFFFFF))
    return lax.bitcast_convert_type(u, jnp.float32)


def _popcnt(m):
    return plsc.all_reduce_population_count(m)


def _count(m):
    return jnp.sum(jnp.where(m, 1, 0))


def _select_kth_key(key_ref, nv, k):
    k_splat = jnp.full((16,), k, jnp.int32)

    def bit_step(i, t):
        sh = (jnp.uint32(31) - i.astype(jnp.uint32))
        cand = t | jnp.full((16,), jnp.uint32(1) << sh, jnp.uint32)

        @plsc.parallel_loop(0, nv, unroll=4, carry=jnp.zeros((16,), jnp.int32))
        def cnt(j, acc):
            kv = key_ref[pl.ds(j * 16, 16)]
            return acc + _popcnt(kv >= cand)

        return jnp.where(cnt >= k_splat, cand, t)

    return lax.fori_loop(0, 32, bit_step, jnp.zeros((16,), jnp.uint32))


def _body(x_hbm, o_hbm, rowbuf, rowbuf2, zbuf, bmax, smax, skey, sv_val,
          sv_id, sv_key, s2_id, c_val, c_idx, c_key, eq_idx, st_val, st_idx,
          st_val2, st_idx2, isem, osem):
    wid = lax.axis_index("s") * _NC + lax.axis_index("c")
    iota = jnp.arange(16, dtype=jnp.int32)
    zero16f = jnp.zeros((16,), jnp.float32)

    @plsc.parallel_loop(0, _N // 16, unroll=8)
    def _(i):
        zbuf[pl.ds(i * 16, 16)] = zero16f

    def select_row(row, rbuf, stv, sti):
        @plsc.parallel_loop(0, _NBV, unroll=4)
        def _(g):
            base = g * 256
            m = rbuf[pl.ds(base, 16)]
            for j in range(1, 16):
                m = jnp.maximum(m, rbuf[pl.ds(base + 16 * j, 16)])
            bmax[pl.ds(g * 16, 16)] = m

        @plsc.parallel_loop(0, _NSV, unroll=2)
        def _(h):
            base = h * 256
            m = bmax[pl.ds(base, 16)]
            for j in range(1, 16):
                m = jnp.maximum(m, bmax[pl.ds(base + 16 * j, 16)])
            smax[pl.ds(h * 16, 16)] = m

        for h in range(_NSV):
            skey[pl.ds(h * 16, 16)] = _keyify(smax[pl.ds(h * 16, 16)])

        tA = _select_kth_key(skey, _NSV, _K)
        tAf = _unkey(tA)

        def sbody(gg, ptr):
            vs, ms, cs = [], [], []
            for u in range(8):
                v = bmax[pl.ds((gg * 8 + u) * 16, 16)]
                m = v >= tAf
                vs.append(v)
                ms.append(m)
                cs.append(_count(m))
            for u in range(8):
                plsc.store_compressed(sv_val.at[pl.ds(ptr, 16)], vs[u],
                                      mask=ms[u])
                plsc.store_compressed(sv_id.at[pl.ds(ptr, 16)],
                                      (gg * 8 + u) * 16 + iota, mask=ms[u])
                ptr = ptr + cs[u]
            return ptr

        n1 = lax.fori_loop(0, _NBV // 8, sbody, jnp.int32(0))
        nv1 = (n1 + 15) // 16

        def kbody(j, _):
            sv_key[pl.ds(j * 16, 16)] = _keyify(sv_val[pl.ds(j * 16, 16)])
            return 0

        lax.fori_loop(0, nv1, kbody, 0)
        sv_key[pl.ds(n1, 16)] = jnp.zeros((16,), jnp.uint32)

        t1 = _select_kth_key(sv_key, nv1, _K)
        t1f = _unkey(t1)

        def s2body(j, ptr):
            v = sv_val[pl.ds(j * 16, 16)]
            ids = sv_id[pl.ds(j * 16, 16)]
            m = (v >= t1f) & ((j * 16 + iota) < n1)
            plsc.store_compressed(s2_id.at[pl.ds(ptr, 16)], ids, mask=m)
            return ptr + _count(m)

        n2 = lax.fori_loop(0, nv1, s2body, jnp.int32(0))
        s2_id[pl.ds(n2, 16)] = jnp.zeros((16,), jnp.int32)
        nb2 = (n2 + 15) // 16

        def cbody(j, ptr):
            ids = s2_id[pl.ds(j * 16, 16)]
            valid = (j * 16 + iota) < n2
            base = (ids >> 4) * 256 + (ids & 15)
            gv, gi, ms, cs = [], [], [], []
            for jj in range(16):
                idxv = base + 16 * jj
                vals = plsc.load_gather(rbuf, [idxv])
                m = (vals >= t1f) & valid
                gv.append(vals)
                gi.append(idxv)
                ms.append(m)
                cs.append(_count(m))
            for jj in range(16):
                plsc.store_compressed(c_val.at[pl.ds(ptr, 16)], gv[jj],
                                      mask=ms[jj])
                plsc.store_compressed(c_idx.at[pl.ds(ptr, 16)], gi[jj],
                                      mask=ms[jj])
                ptr = jnp.minimum(ptr + cs[jj], _CCAP)
            return ptr

        nc = lax.fori_loop(0, nb2, cbody, jnp.int32(0))
        nvc = (nc + 15) // 16

        def ckbody(j, _):
            c_key[pl.ds(j * 16, 16)] = _keyify(c_val[pl.ds(j * 16, 16)])
            return 0

        lax.fori_loop(0, nvc, ckbody, 0)
        c_key[pl.ds(nc, 16)] = jnp.zeros((16,), jnp.uint32)

        tkey = _select_kth_key(c_key, nvc, _K)

        def cgbody(j, cnt):
            return cnt + _popcnt(c_key[pl.ds(j * 16, 16)] > tkey)

        cgv = lax.fori_loop(0, nvc, cgbody, jnp.zeros((16,), jnp.int32))
        need = jnp.full((16,), _K, jnp.int32) - cgv

        def eqbody(j, ptr):
            m = c_key[pl.ds(j * 16, 16)] == tkey
            plsc.store_compressed(eq_idx.at[pl.ds(ptr, 16)],
                                  c_idx[pl.ds(j * 16, 16)], mask=m)
            return ptr + _count(m)

        ne = lax.fori_loop(0, nvc, eqbody, jnp.int32(0))
        eq_idx[pl.ds(ne, 16)] = jnp.full((16,), jnp.int32(0x7FFFFFFF))
        nve = (ne + 15) // 16

        def lstep(i, L):
            cand = L | jnp.full((16,), jnp.int32(1) << (14 - i), jnp.int32)

            def lcnt(j, cnt):
                return cnt + _popcnt(eq_idx[pl.ds(j * 16, 16)] < cand)

            cnt = lax.fori_loop(0, nve, lcnt, jnp.zeros((16,), jnp.int32))
            return jnp.where(cnt <= need - 1, cand, L)

        L = lax.fori_loop(0, 15, lstep, jnp.zeros((16,), jnp.int32))

        def keepbody(j, ptr):
            kv = c_key[pl.ds(j * 16, 16)]
            iv = c_idx[pl.ds(j * 16, 16)]
            m = (kv > tkey) | ((kv == tkey) & (iv <= L))
            plsc.store_compressed(stv.at[pl.ds(ptr, 16)],
                                  c_val[pl.ds(j * 16, 16)], mask=m)
            plsc.store_compressed(sti.at[pl.ds(ptr, 16)], iv, mask=m)
            return ptr + _count(m)

        lax.fori_loop(0, nvc, keepbody, jnp.int32(0))

    row0 = wid * _RPW
    bufs = [rowbuf, rowbuf2]
    sts = [(st_val, st_idx), (st_val2, st_idx2)]
    ih = pltpu.async_copy(x_hbm.at[row0], bufs[0], isem)
    oh = None
    for r in range(_RPW):
        ih.wait()
        if r + 1 < _RPW:
            ih = pltpu.async_copy(x_hbm.at[row0 + r + 1], bufs[(r + 1) % 2],
                                  isem)
        stv, sti = sts[r % 2]
        select_row(row0 + r, bufs[r % 2], stv, sti)
        if oh is not None:
            oh.wait()
            _, psti = sts[(r + 1) % 2]
            for j in range(_K // 16):
                plsc.store_scatter(zbuf, [psti[pl.ds(j * 16, 16)]], zero16f)
        for j in range(_K // 16):
            plsc.store_scatter(zbuf, [sti[pl.ds(j * 16, 16)]],
                               stv[pl.ds(j * 16, 16)])
        oh = pltpu.async_copy(zbuf, o_hbm.at[row0 + r], osem)
    oh.wait()


_sc_call = functools.partial(
    pl.kernel,
    mesh=plsc.VectorSubcoreMesh(core_axis_name="c", subcore_axis_name="s"),
    compiler_params=pltpu.CompilerParams(needs_layout_passes=False),
    out_type=jax.ShapeDtypeStruct((_ROWS, _N), jnp.float32),
    scratch_types=[
        pltpu.VMEM((_N,), jnp.float32),
        pltpu.VMEM((_N,), jnp.float32),
        pltpu.VMEM((_N,), jnp.float32),
        pltpu.VMEM((_N // 16,), jnp.float32),
        pltpu.VMEM((_NBV,), jnp.float32),
        pltpu.VMEM((_NBV,), jnp.uint32),
        pltpu.VMEM((_SVCAP + 16,), jnp.float32),
        pltpu.VMEM((_SVCAP + 16,), jnp.int32),
        pltpu.VMEM((_SVCAP + 16,), jnp.uint32),
        pltpu.VMEM((_SVCAP + 16,), jnp.int32),
        pltpu.VMEM((_CCAP + 16,), jnp.float32),
        pltpu.VMEM((_CCAP + 16,), jnp.int32),
        pltpu.VMEM((_CCAP + 16,), jnp.uint32),
        pltpu.VMEM((_CCAP + 16,), jnp.int32),
        pltpu.VMEM((_K + 16,), jnp.float32),
        pltpu.VMEM((_K + 16,), jnp.int32),
        pltpu.VMEM((_K + 16,), jnp.float32),
        pltpu.VMEM((_K + 16,), jnp.int32),
        pltpu.SemaphoreType.DMA,
        pltpu.SemaphoreType.DMA,
    ],
)(_body)


def kernel(x):
    return _sc_call(x)


# speedup vs baseline: 17.4687x; 1.0297x over previous
import functools

import jax
import jax.numpy as jnp
from jax import lax
from jax.experimental import pallas as pl
from jax.experimental.pallas import tpu as pltpu
from jax.experimental.pallas import tpu_sc as plsc

_K = 64
_N = 32768
_ROWS = 128
_NC = 2
_NS = 16
_NW = _NC * _NS
_RPW = _ROWS // _NW
_NBV = _N // 256
_NSV = _NBV // 16
_SVCAP = 2048
_CCAP = 4096


def _keyify(v):
    u = lax.bitcast_convert_type(v, jnp.uint32)
    return u ^ ((u >> jnp.uint32(31)) * jnp.uint32(0x7FFFFFFF)
                + jnp.uint32(0x80000000))


def _unkey(key):
    pos = key >> jnp.uint32(31)
    u = key ^ (jnp.uint32(0x80000000)
               + (jnp.uint32(1) - pos) * jnp.uint32(0x7FFFFFFF))
    return lax.bitcast_convert_type(u, jnp.float32)


def _popcnt(m):
    return plsc.all_reduce_population_count(m)


def _count(m):
    return _popcnt(m)[0]


def _select_kth_key(key_ref, nv, k):
    k_splat = jnp.full((16,), k, jnp.int32)

    def bit_step(i, t):
        sh = (jnp.uint32(31) - i.astype(jnp.uint32))
        cand = t | jnp.full((16,), jnp.uint32(1) << sh, jnp.uint32)

        @plsc.parallel_loop(0, nv, unroll=4, carry=jnp.zeros((16,), jnp.int32))
        def cnt(j, acc):
            kv = key_ref[pl.ds(j * 16, 16)]
            return acc + _popcnt(kv >= cand)

        return jnp.where(cnt >= k_splat, cand, t)

    return lax.fori_loop(0, 32, bit_step, jnp.zeros((16,), jnp.uint32))


def _body(x_hbm, o_hbm, rowbuf, rowbuf2, zbuf, bmax, smax, skey, sv_val,
          sv_id, sv_key, s2_id, c_val, c_idx, c_key, eq_idx, st_val, st_idx,
          st_val2, st_idx2, isem, osem):
    wid = lax.axis_index("s") * _NC + lax.axis_index("c")
    iota = jnp.arange(16, dtype=jnp.int32)
    zero16f = jnp.zeros((16,), jnp.float32)

    row0 = wid * _RPW
    bufs = [rowbuf, rowbuf2]
    ih = pltpu.async_copy(x_hbm.at[row0], bufs[0], isem)

    @plsc.parallel_loop(0, _N // 16, unroll=8)
    def _(i):
        zbuf[pl.ds(i * 16, 16)] = zero16f

    def select_row(row, rbuf, stv, sti):
        @plsc.parallel_loop(0, _NBV, unroll=4)
        def _(g):
            base = g * 256
            m = rbuf[pl.ds(base, 16)]
            for j in range(1, 16):
                m = jnp.maximum(m, rbuf[pl.ds(base + 16 * j, 16)])
            bmax[pl.ds(g * 16, 16)] = m

        @plsc.parallel_loop(0, _NSV, unroll=2)
        def _(h):
            base = h * 256
            m = bmax[pl.ds(base, 16)]
            for j in range(1, 16):
                m = jnp.maximum(m, bmax[pl.ds(base + 16 * j, 16)])
            smax[pl.ds(h * 16, 16)] = m

        for h in range(_NSV):
            skey[pl.ds(h * 16, 16)] = _keyify(smax[pl.ds(h * 16, 16)])

        tA = _select_kth_key(skey, _NSV, _K)
        tAf = _unkey(tA)

        def sbody(gg, ptr):
            vs, ms, cs = [], [], []
            for u in range(8):
                v = bmax[pl.ds((gg * 8 + u) * 16, 16)]
                m = v >= tAf
                vs.append(v)
                ms.append(m)
                cs.append(_count(m))
            for u in range(8):
                plsc.store_compressed(sv_val.at[pl.ds(ptr, 16)], vs[u],
                                      mask=ms[u])
                plsc.store_compressed(sv_id.at[pl.ds(ptr, 16)],
                                      (gg * 8 + u) * 16 + iota, mask=ms[u])
                ptr = ptr + cs[u]
            return ptr

        n1 = lax.fori_loop(0, _NBV // 8, sbody, jnp.int32(0))
        nv1 = (n1 + 15) // 16

        def kbody(j, _):
            sv_key[pl.ds(j * 16, 16)] = _keyify(sv_val[pl.ds(j * 16, 16)])
            return 0

        lax.fori_loop(0, nv1, kbody, 0)
        sv_key[pl.ds(n1, 16)] = jnp.zeros((16,), jnp.uint32)

        t1 = _select_kth_key(sv_key, nv1, _K)
        t1f = _unkey(t1)

        def s2body(j, ptr):
            v = sv_val[pl.ds(j * 16, 16)]
            ids = sv_id[pl.ds(j * 16, 16)]
            m = (v >= t1f) & ((j * 16 + iota) < n1)
            plsc.store_compressed(s2_id.at[pl.ds(ptr, 16)], ids, mask=m)
            return ptr + _count(m)

        n2 = lax.fori_loop(0, nv1, s2body, jnp.int32(0))
        s2_id[pl.ds(n2, 16)] = jnp.zeros((16,), jnp.int32)
        nb2 = (n2 + 15) // 16

        def cbody(j, ptr):
            ids = s2_id[pl.ds(j * 16, 16)]
            valid = (j * 16 + iota) < n2
            base = (ids >> 4) * 256 + (ids & 15)
            gv, gi, ms, cs = [], [], [], []
            for jj in range(16):
                idxv = base + 16 * jj
                vals = plsc.load_gather(rbuf, [idxv])
                m = (vals >= t1f) & valid
                gv.append(vals)
                gi.append(idxv)
                ms.append(m)
                cs.append(_count(m))
            for jj in range(16):
                plsc.store_compressed(c_val.at[pl.ds(ptr, 16)], gv[jj],
                                      mask=ms[jj])
                plsc.store_compressed(c_idx.at[pl.ds(ptr, 16)], gi[jj],
                                      mask=ms[jj])
                ptr = jnp.minimum(ptr + cs[jj], _CCAP)
            return ptr

        nc = lax.fori_loop(0, nb2, cbody, jnp.int32(0))
        nvc = (nc + 15) // 16

        def ckbody(j, _):
            c_key[pl.ds(j * 16, 16)] = _keyify(c_val[pl.ds(j * 16, 16)])
            return 0

        lax.fori_loop(0, nvc, ckbody, 0)
        c_key[pl.ds(nc, 16)] = jnp.zeros((16,), jnp.uint32)

        tkey = _select_kth_key(c_key, nvc, _K)

        def cgbody(j, cnt):
            return cnt + _popcnt(c_key[pl.ds(j * 16, 16)] > tkey)

        cgv = lax.fori_loop(0, nvc, cgbody, jnp.zeros((16,), jnp.int32))
        need = jnp.full((16,), _K, jnp.int32) - cgv

        def eqbody(j, ptr):
            m = c_key[pl.ds(j * 16, 16)] == tkey
            plsc.store_compressed(eq_idx.at[pl.ds(ptr, 16)],
                                  c_idx[pl.ds(j * 16, 16)], mask=m)
            return ptr + _count(m)

        ne = lax.fori_loop(0, nvc, eqbody, jnp.int32(0))
        eq_idx[pl.ds(ne, 16)] = jnp.full((16,), jnp.int32(0x7FFFFFFF))
        nve = (ne + 15) // 16

        def lstep(i, L):
            cand = L | jnp.full((16,), jnp.int32(1) << (14 - i), jnp.int32)

            def lcnt(j, cnt):
                return cnt + _popcnt(eq_idx[pl.ds(j * 16, 16)] < cand)

            cnt = lax.fori_loop(0, nve, lcnt, jnp.zeros((16,), jnp.int32))
            return jnp.where(cnt <= need - 1, cand, L)

        L = lax.fori_loop(0, 15, lstep, jnp.zeros((16,), jnp.int32))

        def keepbody(j, ptr):
            kv = c_key[pl.ds(j * 16, 16)]
            iv = c_idx[pl.ds(j * 16, 16)]
            m = (kv > tkey) | ((kv == tkey) & (iv <= L))
            plsc.store_compressed(stv.at[pl.ds(ptr, 16)],
                                  c_val[pl.ds(j * 16, 16)], mask=m)
            plsc.store_compressed(sti.at[pl.ds(ptr, 16)], iv, mask=m)
            return ptr + _count(m)

        lax.fori_loop(0, nvc, keepbody, jnp.int32(0))

    sts = [(st_val, st_idx), (st_val2, st_idx2)]
    oh = None
    for r in range(_RPW):
        ih.wait()
        if r + 1 < _RPW:
            ih = pltpu.async_copy(x_hbm.at[row0 + r + 1], bufs[(r + 1) % 2],
                                  isem)
        stv, sti = sts[r % 2]
        select_row(row0 + r, bufs[r % 2], stv, sti)
        if oh is not None:
            oh.wait()
            _, psti = sts[(r + 1) % 2]
            for j in range(_K // 16):
                plsc.store_scatter(zbuf, [psti[pl.ds(j * 16, 16)]], zero16f)
        for j in range(_K // 16):
            plsc.store_scatter(zbuf, [sti[pl.ds(j * 16, 16)]],
                               stv[pl.ds(j * 16, 16)])
        oh = pltpu.async_copy(zbuf, o_hbm.at[row0 + r], osem)
    oh.wait()


_sc_call = functools.partial(
    pl.kernel,
    mesh=plsc.VectorSubcoreMesh(core_axis_name="c", subcore_axis_name="s"),
    compiler_params=pltpu.CompilerParams(needs_layout_passes=False),
    out_type=jax.ShapeDtypeStruct((_ROWS, _N), jnp.float32),
    scratch_types=[
        pltpu.VMEM((_N,), jnp.float32),
        pltpu.VMEM((_N,), jnp.float32),
        pltpu.VMEM((_N,), jnp.float32),
        pltpu.VMEM((_N // 16,), jnp.float32),
        pltpu.VMEM((_NBV,), jnp.float32),
        pltpu.VMEM((_NBV,), jnp.uint32),
        pltpu.VMEM((_SVCAP + 16,), jnp.float32),
        pltpu.VMEM((_SVCAP + 16,), jnp.int32),
        pltpu.VMEM((_SVCAP + 16,), jnp.uint32),
        pltpu.VMEM((_SVCAP + 16,), jnp.int32),
        pltpu.VMEM((_CCAP + 16,), jnp.float32),
        pltpu.VMEM((_CCAP + 16,), jnp.int32),
        pltpu.VMEM((_CCAP + 16,), jnp.uint32),
        pltpu.VMEM((_CCAP + 16,), jnp.int32),
        pltpu.VMEM((_K + 16,), jnp.float32),
        pltpu.VMEM((_K + 16,), jnp.int32),
        pltpu.VMEM((_K + 16,), jnp.float32),
        pltpu.VMEM((_K + 16,), jnp.int32),
        pltpu.SemaphoreType.DMA,
        pltpu.SemaphoreType.DMA,
    ],
)(_body)


def kernel(x):
    return _sc_call(x)


# speedup vs baseline: 17.7225x; 1.0145x over previous
import functools

import jax
import jax.numpy as jnp
from jax import lax
from jax.experimental import pallas as pl
from jax.experimental.pallas import tpu as pltpu
from jax.experimental.pallas import tpu_sc as plsc

_K = 64
_N = 32768
_ROWS = 128
_NC = 2
_NS = 16
_NW = _NC * _NS
_RPW = _ROWS // _NW
_NBV = _N // 256
_NSV = _NBV // 16
_SVCAP = 2048
_CCAP = 4096


def _keyify(v):
    u = lax.bitcast_convert_type(v, jnp.uint32)
    return u ^ ((u >> jnp.uint32(31)) * jnp.uint32(0x7FFFFFFF)
                + jnp.uint32(0x80000000))


def _unkey(key):
    pos = key >> jnp.uint32(31)
    u = key ^ (jnp.uint32(0x80000000)
               + (jnp.uint32(1) - pos) * jnp.uint32(0x7FFFFFFF))
    return lax.bitcast_convert_type(u, jnp.float32)


def _popcnt(m):
    return plsc.all_reduce_population_count(m)


def _count(m):
    return _popcnt(m)[0]


def _select_kth_key(key_ref, nv, k):
    k_splat = jnp.full((16,), k, jnp.int32)

    def bit_step(i, t):
        sh = (jnp.uint32(31) - i.astype(jnp.uint32))
        cand = t | jnp.full((16,), jnp.uint32(1) << sh, jnp.uint32)

        @plsc.parallel_loop(0, nv, unroll=4, carry=jnp.zeros((16,), jnp.int32))
        def cnt(j, acc):
            kv = key_ref[pl.ds(j * 16, 16)]
            return acc + _popcnt(kv >= cand)

        return jnp.where(cnt >= k_splat, cand, t)

    return lax.fori_loop(0, 32, bit_step, jnp.zeros((16,), jnp.uint32))


def _body(x_hbm, o_hbm, rowbuf, rowbuf2, zbuf, bmax, smax, skey,
          sv_id, sv_key, s2_id, c_idx, c_key, eq_idx, st_val, st_idx,
          st_val2, st_idx2, isem, osem):
    wid = lax.axis_index("s") * _NC + lax.axis_index("c")
    iota = jnp.arange(16, dtype=jnp.int32)
    zero16f = jnp.zeros((16,), jnp.float32)

    row0 = wid * _RPW
    bufs = [rowbuf, rowbuf2]
    ih = pltpu.async_copy(x_hbm.at[row0], bufs[0], isem)

    @plsc.parallel_loop(0, _N // 16, unroll=8)
    def _(i):
        zbuf[pl.ds(i * 16, 16)] = zero16f

    def select_row(row, rbuf, stv, sti):
        @plsc.parallel_loop(0, _NBV, unroll=4)
        def _(g):
            base = g * 256
            m = rbuf[pl.ds(base, 16)]
            for j in range(1, 16):
                m = jnp.maximum(m, rbuf[pl.ds(base + 16 * j, 16)])
            bmax[pl.ds(g * 16, 16)] = m

        @plsc.parallel_loop(0, _NSV, unroll=2)
        def _(h):
            base = h * 256
            m = bmax[pl.ds(base, 16)]
            for j in range(1, 16):
                m = jnp.maximum(m, bmax[pl.ds(base + 16 * j, 16)])
            smax[pl.ds(h * 16, 16)] = m

        for h in range(_NSV):
            skey[pl.ds(h * 16, 16)] = _keyify(smax[pl.ds(h * 16, 16)])

        tA = _select_kth_key(skey, _NSV, _K)

        def sbody(gg, ptr):
            vs, ms, cs = [], [], []
            for u in range(8):
                kv = _keyify(bmax[pl.ds((gg * 8 + u) * 16, 16)])
                m = kv >= tA
                vs.append(kv)
                ms.append(m)
                cs.append(_count(m))
            for u in range(8):
                plsc.store_compressed(sv_key.at[pl.ds(ptr, 16)], vs[u],
                                      mask=ms[u])
                plsc.store_compressed(sv_id.at[pl.ds(ptr, 16)],
                                      (gg * 8 + u) * 16 + iota, mask=ms[u])
                ptr = ptr + cs[u]
            return ptr

        n1 = lax.fori_loop(0, _NBV // 8, sbody, jnp.int32(0))
        nv1 = (n1 + 15) // 16
        sv_key[pl.ds(n1, 16)] = jnp.zeros((16,), jnp.uint32)

        t1 = _select_kth_key(sv_key, nv1, _K)

        def s2body(j, ptr):
            kv = sv_key[pl.ds(j * 16, 16)]
            ids = sv_id[pl.ds(j * 16, 16)]
            m = (kv >= t1) & ((j * 16 + iota) < n1)
            plsc.store_compressed(s2_id.at[pl.ds(ptr, 16)], ids, mask=m)
            return ptr + _count(m)

        n2 = lax.fori_loop(0, nv1, s2body, jnp.int32(0))
        s2_id[pl.ds(n2, 16)] = jnp.zeros((16,), jnp.int32)
        nb2 = (n2 + 15) // 16

        def cbody(j, ptr):
            ids = s2_id[pl.ds(j * 16, 16)]
            valid = (j * 16 + iota) < n2
            base = (ids >> 4) * 256 + (ids & 15)
            gv, gi, ms, cs = [], [], [], []
            for jj in range(16):
                idxv = base + 16 * jj
                kv = _keyify(plsc.load_gather(rbuf, [idxv]))
                m = (kv >= t1) & valid
                gv.append(kv)
                gi.append(idxv)
                ms.append(m)
                cs.append(_count(m))
            for jj in range(16):
                plsc.store_compressed(c_key.at[pl.ds(ptr, 16)], gv[jj],
                                      mask=ms[jj])
                plsc.store_compressed(c_idx.at[pl.ds(ptr, 16)], gi[jj],
                                      mask=ms[jj])
                ptr = jnp.minimum(ptr + cs[jj], _CCAP)
            return ptr

        nc = lax.fori_loop(0, nb2, cbody, jnp.int32(0))
        nvc = (nc + 15) // 16
        c_key[pl.ds(nc, 16)] = jnp.zeros((16,), jnp.uint32)

        tkey = _select_kth_key(c_key, nvc, _K)

        def cgbody(j, cnt):
            return cnt + _popcnt(c_key[pl.ds(j * 16, 16)] > tkey)

        cgv = lax.fori_loop(0, nvc, cgbody, jnp.zeros((16,), jnp.int32))
        need = jnp.full((16,), _K, jnp.int32) - cgv

        def eqbody(j, ptr):
            m = c_key[pl.ds(j * 16, 16)] == tkey
            plsc.store_compressed(eq_idx.at[pl.ds(ptr, 16)],
                                  c_idx[pl.ds(j * 16, 16)], mask=m)
            return ptr + _count(m)

        ne = lax.fori_loop(0, nvc, eqbody, jnp.int32(0))
        eq_idx[pl.ds(ne, 16)] = jnp.full((16,), jnp.int32(0x7FFFFFFF))
        nve = (ne + 15) // 16

        def lstep(i, L):
            cand = L | jnp.full((16,), jnp.int32(1) << (14 - i), jnp.int32)

            def lcnt(j, cnt):
                return cnt + _popcnt(eq_idx[pl.ds(j * 16, 16)] < cand)

            cnt = lax.fori_loop(0, nve, lcnt, jnp.zeros((16,), jnp.int32))
            return jnp.where(cnt <= need - 1, cand, L)

        L = lax.fori_loop(0, 15, lstep, jnp.zeros((16,), jnp.int32))

        def keepbody(j, ptr):
            kv = c_key[pl.ds(j * 16, 16)]
            iv = c_idx[pl.ds(j * 16, 16)]
            m = (kv > tkey) | ((kv == tkey) & (iv <= L))
            plsc.store_compressed(stv.at[pl.ds(ptr, 16)], _unkey(kv), mask=m)
            plsc.store_compressed(sti.at[pl.ds(ptr, 16)], iv, mask=m)
            return ptr + _count(m)

        lax.fori_loop(0, nvc, keepbody, jnp.int32(0))

    sts = [(st_val, st_idx), (st_val2, st_idx2)]
    oh = None
    for r in range(_RPW):
        ih.wait()
        if r + 1 < _RPW:
            ih = pltpu.async_copy(x_hbm.at[row0 + r + 1], bufs[(r + 1) % 2],
                                  isem)
        stv, sti = sts[r % 2]
        select_row(row0 + r, bufs[r % 2], stv, sti)
        if oh is not None:
            oh.wait()
            _, psti = sts[(r + 1) % 2]
            for j in range(_K // 16):
                plsc.store_scatter(zbuf, [psti[pl.ds(j * 16, 16)]], zero16f)
        for j in range(_K // 16):
            plsc.store_scatter(zbuf, [sti[pl.ds(j * 16, 16)]],
                               stv[pl.ds(j * 16, 16)])
        oh = pltpu.async_copy(zbuf, o_hbm.at[row0 + r], osem)
    oh.wait()


_sc_call = functools.partial(
    pl.kernel,
    mesh=plsc.VectorSubcoreMesh(core_axis_name="c", subcore_axis_name="s"),
    compiler_params=pltpu.CompilerParams(needs_layout_passes=False),
    out_type=jax.ShapeDtypeStruct((_ROWS, _N), jnp.float32),
    scratch_types=[
        pltpu.VMEM((_N,), jnp.float32),
        pltpu.VMEM((_N,), jnp.float32),
        pltpu.VMEM((_N,), jnp.float32),
        pltpu.VMEM((_N // 16,), jnp.float32),
        pltpu.VMEM((_NBV,), jnp.float32),
        pltpu.VMEM((_NBV,), jnp.uint32),
        pltpu.VMEM((_SVCAP + 16,), jnp.int32),
        pltpu.VMEM((_SVCAP + 16,), jnp.uint32),
        pltpu.VMEM((_SVCAP + 16,), jnp.int32),
        pltpu.VMEM((_CCAP + 16,), jnp.int32),
        pltpu.VMEM((_CCAP + 16,), jnp.uint32),
        pltpu.VMEM((_CCAP + 16,), jnp.int32),
        pltpu.VMEM((_K + 16,), jnp.float32),
        pltpu.VMEM((_K + 16,), jnp.int32),
        pltpu.VMEM((_K + 16,), jnp.float32),
        pltpu.VMEM((_K + 16,), jnp.int32),
        pltpu.SemaphoreType.DMA,
        pltpu.SemaphoreType.DMA,
    ],
)(_body)


def kernel(x):
    return _sc_call(x)
